# baseline (speedup 1.0000x reference)
import jax
import jax.numpy as jnp
from jax import lax
from jax.experimental import pallas as pl
from jax.experimental.pallas import tpu as pltpu
from jax.experimental.pallas import tpu_sc as plsc

N_NODES = 10000
N_EDGES = 320000
D = 128

NUM_CORES = 2
NUM_SUBCORES = 16
NW = NUM_CORES * NUM_SUBCORES
EDGES_PER_WORKER = N_EDGES // NW
CHUNK = 40
NCHUNKS = EDGES_PER_WORKER // CHUNK
DRAIN_ROWS = 40
DRAIN_CHUNKS = N_NODES // DRAIN_ROWS
NLANE = 16


def _node_proj_body(x_ref, w_ref, b_ref, h_ref, q_ref, kv_ref):
    p = jnp.dot(x_ref[...], w_ref[...], preferred_element_type=jnp.float32)
    p = p + b_ref[...]
    h_ref[...] = p[:, 0 * D:1 * D]
    q_ref[...] = p[:, 1 * D:2 * D]
    kv_ref[...] = p[:, 2 * D:4 * D]


def _node_proj(x, w, b):
    blk = 1000
    grid = N_NODES // blk
    return pl.pallas_call(
        _node_proj_body,
        grid=(grid,),
        in_specs=[
            pl.BlockSpec((blk, D), lambda i: (i, 0)),
            pl.BlockSpec((D, 4 * D), lambda i: (0, 0)),
            pl.BlockSpec((1, 4 * D), lambda i: (0, 0)),
        ],
        out_specs=[
            pl.BlockSpec((blk, D), lambda i: (i, 0)),
            pl.BlockSpec((blk, D), lambda i: (i, 0)),
            pl.BlockSpec((blk, 2 * D), lambda i: (i, 0)),
        ],
        out_shape=[
            jax.ShapeDtypeStruct((N_NODES, D), jnp.float32),
            jax.ShapeDtypeStruct((N_NODES, D), jnp.float32),
            jax.ShapeDtypeStruct((N_NODES, 2 * D), jnp.float32),
        ],
    )(x, w, b.reshape(1, 4 * D))


def _edge_proj_body(ef_ref, we_ref, be_ref, o_ref):
    o_ref[...] = jnp.dot(ef_ref[...], we_ref[...],
                         preferred_element_type=jnp.float32) + be_ref[...]


def _edge_proj(ef, we, be):
    blk = 8000
    grid = N_EDGES // blk
    return pl.pallas_call(
        _edge_proj_body,
        grid=(grid,),
        in_specs=[
            pl.BlockSpec((blk, ef.shape[1]), lambda i: (i, 0)),
            pl.BlockSpec((ef.shape[1], D), lambda i: (0, 0)),
            pl.BlockSpec((1, D), lambda i: (0, 0)),
        ],
        out_specs=pl.BlockSpec((blk, D), lambda i: (i, 0)),
        out_shape=jax.ShapeDtypeStruct((N_EDGES, D), jnp.float32),
    )(ef, we, be.reshape(1, D))


def _sc_body(q_hbm, kv_hbm, ep_hbm, s_hbm, r_hbm, out_hbm,
             sidx, ridx, qb0, kvb0, qb1, kvb1, eb,
             acc, sem0, sem1):
    cid = lax.axis_index("c")
    sid = lax.axis_index("s")
    wid = cid * NUM_SUBCORES + sid

    @pl.loop(0, DRAIN_ROWS)
    def _(i):
        for j in range(D // NLANE):
            eb[i, pl.ds(j * NLANE, NLANE)] = jnp.zeros((NLANE,), jnp.float32)

    for t in range(-(-DRAIN_CHUNKS // NUM_SUBCORES)):
        c = sid + t * NUM_SUBCORES

        @pl.when(c < DRAIN_CHUNKS)
        def _():
            off = pl.multiple_of(c * DRAIN_ROWS, 8)
            pltpu.sync_copy(eb.at[pl.ds(0, DRAIN_ROWS)],
                            acc.at[pl.ds(off, DRAIN_ROWS)])

    plsc.subcore_barrier()

    bufs = ((qb0, kvb0, sem0), (qb1, kvb1, sem1))

    @pl.loop(0, NCHUNKS, step=2)
    def _(i):
        row = pl.multiple_of(wid * (EDGES_PER_WORKER // CHUNK) + i, 2)
        base = pl.multiple_of((wid * EDGES_PER_WORKER + i * CHUNK), 8)
        pltpu.sync_copy(s_hbm.at[pl.ds(row, 2)], sidx)
        pltpu.sync_copy(r_hbm.at[pl.ds(row, 2)], ridx)
        de = pltpu.async_copy(ep_hbm.at[pl.ds(base, 2 * CHUNK)], eb, sem0)
        copies = []
        for b in range(2):
            qb, kvb, sem = bufs[b]
            copies.append((
                pltpu.async_copy(q_hbm.at[ridx.at[b]], qb, sem),
                pltpu.async_copy(kv_hbm.at[sidx.at[b]], kvb, sem),
            ))
        de.wait()
        for b in range(2):
            qb, kvb, sem = bufs[b]
            dq, dkv = copies[b]
            dq.wait()
            dkv.wait()

            @pl.loop(0, CHUNK)
            def _(e):
                for j in range(D // NLANE):
                    sl = pl.ds(j * NLANE, NLANE)
                    x = qb[e, sl] + kvb[e, sl] + eb[b * CHUNK + e, sl]
                    eta = 1.0 / (1.0 + jnp.exp(-x))
                    qb[e, sl] = eta * kvb[e, pl.ds(D + j * NLANE, NLANE)]

            pltpu.sync_copy(qb, acc.at[ridx.at[b]], add=True)

    plsc.subcore_barrier()

    for t in range(-(-DRAIN_CHUNKS // NUM_SUBCORES)):
        c = sid + t * NUM_SUBCORES

        @pl.when(c < DRAIN_CHUNKS)
        def _():
            off = pl.multiple_of(c * DRAIN_ROWS, 8)
            rows = pl.ds(off, DRAIN_ROWS)
            pltpu.sync_copy(acc.at[rows], eb.at[pl.ds(0, DRAIN_ROWS)])
            pltpu.sync_copy(eb.at[pl.ds(0, DRAIN_ROWS)], out_hbm.at[cid, rows])


def _sc_gather_scatter(q, kv, ep, senders, receivers):
    mesh = plsc.VectorSubcoreMesh(core_axis_name="c", subcore_axis_name="s")
    kern = pl.kernel(
        _sc_body,
        mesh=mesh,
        out_type=jax.ShapeDtypeStruct((NUM_CORES, N_NODES, D), jnp.float32),
        scratch_types=[
            pltpu.VMEM((2, CHUNK), jnp.int32),
            pltpu.VMEM((2, CHUNK), jnp.int32),
            pltpu.VMEM((CHUNK, D), jnp.float32),
            pltpu.VMEM((CHUNK, 2 * D), jnp.float32),
            pltpu.VMEM((CHUNK, D), jnp.float32),
            pltpu.VMEM((CHUNK, 2 * D), jnp.float32),
            pltpu.VMEM((2 * CHUNK, D), jnp.float32),
            pltpu.VMEM_SHARED((N_NODES, D), jnp.float32),
            pltpu.SemaphoreType.DMA,
            pltpu.SemaphoreType.DMA,
        ],
    )
    return kern(q, kv, ep,
                senders.reshape(N_EDGES // CHUNK, CHUNK),
                receivers.reshape(N_EDGES // CHUNK, CHUNK))


def _combine_body(h_ref, p_ref, o_ref):
    o_ref[...] = h_ref[...] + p_ref[0] + p_ref[1]


def _combine(h, partials):
    blk = 1000
    grid = N_NODES // blk
    return pl.pallas_call(
        _combine_body,
        grid=(grid,),
        in_specs=[
            pl.BlockSpec((blk, D), lambda i: (i, 0)),
            pl.BlockSpec((NUM_CORES, blk, D), lambda i: (0, i, 0)),
        ],
        out_specs=pl.BlockSpec((blk, D), lambda i: (i, 0)),
        out_shape=jax.ShapeDtypeStruct((N_NODES, D), jnp.float32),
    )(h, partials)


@jax.jit
def kernel(node_features, senders, receivers, edge_features,
           W_kernel, W_bias, We_kernel, We_bias):
    h, q, kv = _node_proj(node_features, W_kernel, W_bias)
    ep = _edge_proj(edge_features, We_kernel, We_bias)
    partials = _sc_gather_scatter(q, kv, ep, senders, receivers)
    return _combine(h, partials)


# speedup vs baseline: 2.5740x; 2.5740x over previous
import jax
import jax.numpy as jnp
from jax import lax
from jax.experimental import pallas as pl
from jax.experimental.pallas import tpu as pltpu
from jax.experimental.pallas import tpu_sc as plsc

N_NODES = 10000
N_EDGES = 320000
D = 128

NUM_CORES = 2
NUM_SUBCORES = 16
NW = NUM_CORES * NUM_SUBCORES
EDGES_PER_WORKER = N_EDGES // NW
CHUNK = 40
NCHUNKS = EDGES_PER_WORKER // CHUNK
DRAIN_ROWS = 40
DRAIN_CHUNKS = N_NODES // DRAIN_ROWS
NLANE = 16


def _node_proj_body(x_ref, w_ref, b_ref, h_ref, q_ref, k_ref, v_ref):
    p = jnp.dot(x_ref[...], w_ref[...], preferred_element_type=jnp.float32)
    p = p + b_ref[...]
    h_ref[...] = p[:, 0 * D:1 * D]
    q_ref[...] = p[:, 1 * D:2 * D]
    k_ref[...] = p[:, 2 * D:3 * D]
    v_ref[...] = p[:, 3 * D:4 * D]


def _node_proj(x, w, b):
    blk = 1000
    grid = N_NODES // blk
    return pl.pallas_call(
        _node_proj_body,
        grid=(grid,),
        in_specs=[
            pl.BlockSpec((blk, D), lambda i: (i, 0)),
            pl.BlockSpec((D, 4 * D), lambda i: (0, 0)),
            pl.BlockSpec((1, 4 * D), lambda i: (0, 0)),
        ],
        out_specs=[pl.BlockSpec((blk, D), lambda i: (i, 0))] * 4,
        out_shape=[jax.ShapeDtypeStruct((N_NODES, D), jnp.float32)] * 4,
    )(x, w, b.reshape(1, 4 * D))


def _edge_proj_body(ef_ref, we_ref, be_ref, o_ref):
    o_ref[...] = jnp.dot(ef_ref[...], we_ref[...],
                         preferred_element_type=jnp.float32) + be_ref[...]


def _edge_proj(ef, we, be):
    blk = 8000
    grid = N_EDGES // blk
    return pl.pallas_call(
        _edge_proj_body,
        grid=(grid,),
        in_specs=[
            pl.BlockSpec((blk, ef.shape[1]), lambda i: (i, 0)),
            pl.BlockSpec((ef.shape[1], D), lambda i: (0, 0)),
            pl.BlockSpec((1, D), lambda i: (0, 0)),
        ],
        out_specs=pl.BlockSpec((blk, D), lambda i: (i, 0)),
        out_shape=jax.ShapeDtypeStruct((N_EDGES, D), jnp.float32),
    )(ef, we, be.reshape(1, D))


def _sc_body(q_hbm, k_hbm, v_hbm, ep_hbm, s_hbm, r_hbm, out_hbm,
             sidx, ridx, qb0, kb0, vb0, qb1, kb1, vb1, eb,
             acc, sem0, sem1):
    cid = lax.axis_index("c")
    sid = lax.axis_index("s")
    wid = cid * NUM_SUBCORES + sid

    @pl.loop(0, DRAIN_ROWS)
    def _(i):
        for j in range(D // NLANE):
            eb[i, pl.ds(j * NLANE, NLANE)] = jnp.zeros((NLANE,), jnp.float32)

    for t in range(-(-DRAIN_CHUNKS // NUM_SUBCORES)):
        c = sid + t * NUM_SUBCORES

        @pl.when(c < DRAIN_CHUNKS)
        def _():
            off = pl.multiple_of(c * DRAIN_ROWS, 8)
            pltpu.sync_copy(eb.at[pl.ds(0, DRAIN_ROWS)],
                            acc.at[pl.ds(off, DRAIN_ROWS)])

    plsc.subcore_barrier()

    bufs = ((qb0, kb0, vb0, sem0), (qb1, kb1, vb1, sem1))

    @pl.loop(0, NCHUNKS, step=2)
    def _(i):
        row = pl.multiple_of(wid * (EDGES_PER_WORKER // CHUNK) + i, 2)
        base = pl.multiple_of((wid * EDGES_PER_WORKER + i * CHUNK), 8)
        pltpu.sync_copy(s_hbm.at[pl.ds(row, 2)], sidx)
        pltpu.sync_copy(r_hbm.at[pl.ds(row, 2)], ridx)
        de = pltpu.async_copy(ep_hbm.at[pl.ds(base, 2 * CHUNK)], eb, sem0)
        copies = []
        for b in range(2):
            qb, kb, vb, sem = bufs[b]
            copies.append((
                pltpu.async_copy(q_hbm.at[ridx.at[b]], qb, sem),
                pltpu.async_copy(k_hbm.at[sidx.at[b]], kb, sem),
                pltpu.async_copy(v_hbm.at[sidx.at[b]], vb, sem),
            ))
        de.wait()
        for b in range(2):
            qb, kb, vb, sem = bufs[b]
            dq, dk, dv = copies[b]
            dq.wait()
            dk.wait()
            dv.wait()

            @pl.loop(0, CHUNK)
            def _(e):
                for j in range(D // NLANE):
                    sl = pl.ds(j * NLANE, NLANE)
                    x = qb[e, sl] + kb[e, sl] + eb[b * CHUNK + e, sl]
                    eta = 1.0 / (1.0 + jnp.exp(-x))
                    qb[e, sl] = eta * vb[e, sl]

            pltpu.sync_copy(qb, acc.at[ridx.at[b]], add=True)

    plsc.subcore_barrier()

    for t in range(-(-DRAIN_CHUNKS // NUM_SUBCORES)):
        c = sid + t * NUM_SUBCORES

        @pl.when(c < DRAIN_CHUNKS)
        def _():
            off = pl.multiple_of(c * DRAIN_ROWS, 8)
            rows = pl.ds(off, DRAIN_ROWS)
            pltpu.sync_copy(acc.at[rows], eb.at[pl.ds(0, DRAIN_ROWS)])
            pltpu.sync_copy(eb.at[pl.ds(0, DRAIN_ROWS)], out_hbm.at[cid, rows])


def _sc_gather_scatter(q, k, v, ep, senders, receivers):
    mesh = plsc.VectorSubcoreMesh(core_axis_name="c", subcore_axis_name="s")
    kern = pl.kernel(
        _sc_body,
        mesh=mesh,
        out_type=jax.ShapeDtypeStruct((NUM_CORES, N_NODES, D), jnp.float32),
        scratch_types=[
            pltpu.VMEM((2, CHUNK), jnp.int32),
            pltpu.VMEM((2, CHUNK), jnp.int32),
            pltpu.VMEM((CHUNK, D), jnp.float32),
            pltpu.VMEM((CHUNK, D), jnp.float32),
            pltpu.VMEM((CHUNK, D), jnp.float32),
            pltpu.VMEM((CHUNK, D), jnp.float32),
            pltpu.VMEM((CHUNK, D), jnp.float32),
            pltpu.VMEM((CHUNK, D), jnp.float32),
            pltpu.VMEM((2 * CHUNK, D), jnp.float32),
            pltpu.VMEM_SHARED((N_NODES, D), jnp.float32),
            pltpu.SemaphoreType.DMA,
            pltpu.SemaphoreType.DMA,
        ],
    )
    return kern(q, k, v, ep,
                senders.reshape(N_EDGES // CHUNK, CHUNK),
                receivers.reshape(N_EDGES // CHUNK, CHUNK))


def _combine_body(h_ref, p_ref, o_ref):
    o_ref[...] = h_ref[...] + p_ref[0] + p_ref[1]


def _combine(h, partials):
    blk = 1000
    grid = N_NODES // blk
    return pl.pallas_call(
        _combine_body,
        grid=(grid,),
        in_specs=[
            pl.BlockSpec((blk, D), lambda i: (i, 0)),
            pl.BlockSpec((NUM_CORES, blk, D), lambda i: (0, i, 0)),
        ],
        out_specs=pl.BlockSpec((blk, D), lambda i: (i, 0)),
        out_shape=jax.ShapeDtypeStruct((N_NODES, D), jnp.float32),
    )(h, partials)


@jax.jit
def kernel(node_features, senders, receivers, edge_features,
           W_kernel, W_bias, We_kernel, We_bias):
    h, q, k, v = _node_proj(node_features, W_kernel, W_bias)
    ep = _edge_proj(edge_features, We_kernel, We_bias)
    partials = _sc_gather_scatter(q, k, v, ep, senders, receivers)
    return _combine(h, partials)


# speedup vs baseline: 2.8425x; 1.1043x over previous
import jax
import jax.numpy as jnp
from jax import lax
from jax.experimental import pallas as pl
from jax.experimental.pallas import tpu as pltpu
from jax.experimental.pallas import tpu_sc as plsc

N_NODES = 10000
N_EDGES = 320000
D = 128

NUM_CORES = 2
NUM_SUBCORES = 16
NW = NUM_CORES * NUM_SUBCORES
EDGES_PER_WORKER = N_EDGES // NW
CHUNK = 40
NCHUNKS = EDGES_PER_WORKER // CHUNK
BLOCK = 10
NBLOCKS = NCHUNKS // BLOCK
DRAIN_ROWS = 40
DRAIN_CHUNKS = N_NODES // DRAIN_ROWS
NLANE = 16


def _node_proj_body(x_ref, w_ref, b_ref, h_ref, q_ref, k_ref, v_ref):
    p = jnp.dot(x_ref[...], w_ref[...], preferred_element_type=jnp.float32)
    p = p + b_ref[...]
    h_ref[...] = p[:, 0 * D:1 * D]
    q_ref[...] = p[:, 1 * D:2 * D]
    k_ref[...] = p[:, 2 * D:3 * D]
    v_ref[...] = p[:, 3 * D:4 * D]


def _node_proj(x, w, b):
    blk = 1000
    grid = N_NODES // blk
    return pl.pallas_call(
        _node_proj_body,
        grid=(grid,),
        in_specs=[
            pl.BlockSpec((blk, D), lambda i: (i, 0)),
            pl.BlockSpec((D, 4 * D), lambda i: (0, 0)),
            pl.BlockSpec((1, 4 * D), lambda i: (0, 0)),
        ],
        out_specs=[pl.BlockSpec((blk, D), lambda i: (i, 0))] * 4,
        out_shape=[jax.ShapeDtypeStruct((N_NODES, D), jnp.float32)] * 4,
    )(x, w, b.reshape(1, 4 * D))


def _edge_proj_body(ef_ref, we_ref, be_ref, o_ref):
    o_ref[...] = jnp.dot(ef_ref[...], we_ref[...],
                         preferred_element_type=jnp.float32) + be_ref[...]


def _edge_proj(ef, we, be):
    blk = 8000
    grid = N_EDGES // blk
    return pl.pallas_call(
        _edge_proj_body,
        grid=(grid,),
        in_specs=[
            pl.BlockSpec((blk, ef.shape[1]), lambda i: (i, 0)),
            pl.BlockSpec((ef.shape[1], D), lambda i: (0, 0)),
            pl.BlockSpec((1, D), lambda i: (0, 0)),
        ],
        out_specs=pl.BlockSpec((blk, D), lambda i: (i, 0)),
        out_shape=jax.ShapeDtypeStruct((N_EDGES, D), jnp.float32),
    )(ef, we, be.reshape(1, D))


def _sc_body(q_hbm, k_hbm, v_hbm, ep_hbm, s_hbm, r_hbm, out_hbm,
             sidxB, ridxB, qb0, kb0, vb0, qb1, kb1, vb1, eb,
             acc, sem0, sem1, sem2):
    cid = lax.axis_index("c")
    sid = lax.axis_index("s")
    wid = cid * NUM_SUBCORES + sid

    @pl.loop(0, DRAIN_ROWS)
    def _(i):
        for j in range(D // NLANE):
            qb0[i, pl.ds(j * NLANE, NLANE)] = jnp.zeros((NLANE,), jnp.float32)

    for t in range(-(-DRAIN_CHUNKS // NUM_SUBCORES)):
        c = sid + t * NUM_SUBCORES

        @pl.when(c < DRAIN_CHUNKS)
        def _():
            off = pl.multiple_of(c * DRAIN_ROWS, 8)
            pltpu.sync_copy(qb0, acc.at[pl.ds(off, DRAIN_ROWS)])

    plsc.subcore_barrier()

    bufs = ((qb0, kb0, vb0, sem0), (qb1, kb1, vb1, sem1))

    def fire(row, b):
        qb, kb, vb, sem = bufs[b]
        pltpu.async_copy(q_hbm.at[ridxB.at[row]], qb, sem)
        pltpu.async_copy(k_hbm.at[sidxB.at[row]], kb, sem)
        pltpu.async_copy(v_hbm.at[sidxB.at[row]], vb, sem)

    def wait(row, b):
        qb, kb, vb, sem = bufs[b]
        pltpu.make_async_copy(q_hbm.at[ridxB.at[row]], qb, sem).wait()
        pltpu.make_async_copy(k_hbm.at[sidxB.at[row]], kb, sem).wait()
        pltpu.make_async_copy(v_hbm.at[sidxB.at[row]], vb, sem).wait()

    @pl.loop(0, NBLOCKS)
    def _(blk):
        pltpu.sync_copy(s_hbm.at[wid * NBLOCKS + blk], sidxB)
        pltpu.sync_copy(r_hbm.at[wid * NBLOCKS + blk], ridxB)
        fire(0, 0)

        @pl.loop(0, BLOCK, step=2)
        def _(i):
            for b in range(2):
                r = i + b

                @pl.when(r + 1 < BLOCK)
                def _():
                    fire(r + 1, 1 - b)

                qb, kb, vb, sem = bufs[b]
                base = pl.multiple_of(
                    (wid * EDGES_PER_WORKER + (blk * BLOCK + r) * CHUNK), 8)
                de = pltpu.async_copy(ep_hbm.at[pl.ds(base, CHUNK)], eb, sem2)
                wait(r, b)
                de.wait()

                @pl.loop(0, CHUNK)
                def _(e):
                    for j in range(D // NLANE):
                        sl = pl.ds(j * NLANE, NLANE)
                        x = qb[e, sl] + kb[e, sl] + eb[e, sl]
                        eta = 1.0 / (1.0 + jnp.exp(-x))
                        qb[e, sl] = eta * vb[e, sl]

                pltpu.sync_copy(qb, acc.at[ridxB.at[r]], add=True)

    plsc.subcore_barrier()

    for t in range(-(-DRAIN_CHUNKS // NUM_SUBCORES)):
        c = sid + t * NUM_SUBCORES

        @pl.when(c < DRAIN_CHUNKS)
        def _():
            off = pl.multiple_of(c * DRAIN_ROWS, 8)
            rows = pl.ds(off, DRAIN_ROWS)
            pltpu.sync_copy(acc.at[rows], qb0)
            pltpu.sync_copy(qb0, out_hbm.at[cid, rows])


def _sc_gather_scatter(q, k, v, ep, senders, receivers):
    mesh = plsc.VectorSubcoreMesh(core_axis_name="c", subcore_axis_name="s")
    kern = pl.kernel(
        _sc_body,
        mesh=mesh,
        out_type=jax.ShapeDtypeStruct((NUM_CORES, N_NODES, D), jnp.float32),
        scratch_types=[
            pltpu.VMEM((BLOCK, CHUNK), jnp.int32),
            pltpu.VMEM((BLOCK, CHUNK), jnp.int32),
            pltpu.VMEM((CHUNK, D), jnp.float32),
            pltpu.VMEM((CHUNK, D), jnp.float32),
            pltpu.VMEM((CHUNK, D), jnp.float32),
            pltpu.VMEM((CHUNK, D), jnp.float32),
            pltpu.VMEM((CHUNK, D), jnp.float32),
            pltpu.VMEM((CHUNK, D), jnp.float32),
            pltpu.VMEM((CHUNK, D), jnp.float32),
            pltpu.VMEM_SHARED((N_NODES, D), jnp.float32),
            pltpu.SemaphoreType.DMA,
            pltpu.SemaphoreType.DMA,
            pltpu.SemaphoreType.DMA,
        ],
    )
    return kern(q, k, v, ep,
                senders.reshape(NW * NBLOCKS, BLOCK, CHUNK),
                receivers.reshape(NW * NBLOCKS, BLOCK, CHUNK))


def _combine_body(h_ref, p_ref, o_ref):
    o_ref[...] = h_ref[...] + p_ref[0] + p_ref[1]


def _combine(h, partials):
    blk = 1000
    grid = N_NODES // blk
    return pl.pallas_call(
        _combine_body,
        grid=(grid,),
        in_specs=[
            pl.BlockSpec((blk, D), lambda i: (i, 0)),
            pl.BlockSpec((NUM_CORES, blk, D), lambda i: (0, i, 0)),
        ],
        out_specs=pl.BlockSpec((blk, D), lambda i: (i, 0)),
        out_shape=jax.ShapeDtypeStruct((N_NODES, D), jnp.float32),
    )(h, partials)


@jax.jit
def kernel(node_features, senders, receivers, edge_features,
           W_kernel, W_bias, We_kernel, We_bias):
    h, q, k, v = _node_proj(node_features, W_kernel, W_bias)
    ep = _edge_proj(edge_features, We_kernel, We_bias)
    partials = _sc_gather_scatter(q, k, v, ep, senders, receivers)
    return _combine(h, partials)
